# baseline (speedup 1.0000x reference)
import functools

import jax
import jax.numpy as jnp
from jax import lax
from jax.experimental import pallas as pl
from jax.experimental.pallas import tpu as pltpu
from jax.experimental.pallas import tpu_sc as plsc

NC = 2
NS = 16
NW = NC * NS
CHUNK = 128


def _leaky(v):
    return jnp.where(v >= 0, v, 0.01 * v)


def _lin_in_body(x_ref, w_ref, b_ref, h_ref):
    h = jnp.dot(x_ref[...], w_ref[...], preferred_element_type=jnp.float32)
    h_ref[...] = _leaky(h + b_ref[...])


def _lin_in(x, W_in, b_in):
    n, c = x.shape[0], W_in.shape[1]
    return pl.pallas_call(
        _lin_in_body,
        out_shape=jax.ShapeDtypeStruct((n, c), jnp.float32),
    )(x, W_in, b_in.reshape(1, c))


def _make_gather(n_nodes, c, ch_per_w, e_pad):
    mesh = plsc.VectorSubcoreMesh(core_axis_name="c", subcore_axis_name="s")

    @functools.partial(
        pl.kernel,
        out_type=jax.ShapeDtypeStruct((e_pad, c), jnp.float32),
        mesh=mesh,
        scratch_types=[
            pltpu.VMEM((ch_per_w, CHUNK), jnp.int32),
            pltpu.VMEM((8 * CHUNK, c), jnp.float32),
            pltpu.VMEM((8 * CHUNK, c), jnp.float32),
            pltpu.SemaphoreType.DMA,
            pltpu.SemaphoreType.DMA,
            pltpu.SemaphoreType.DMA,
            pltpu.SemaphoreType.DMA,
        ],
        compiler_params=pltpu.CompilerParams(use_tc_tiling_on_sc=False),
    )
    def gather_k(h_hbm, src_hbm, xj_hbm, idx_v, buf_a, buf_b, ga, gb, sa, sb):
        w = lax.axis_index("c") * NS + lax.axis_index("s")
        pltpu.sync_copy(src_hbm.at[pl.ds(w * ch_per_w, ch_per_w)], idx_v)
        n_it = ch_per_w // 16

        def gather_batch(i, b0, buf, gsem):
            for k in range(8):
                pltpu.async_copy(
                    h_hbm.at[idx_v.at[i * 16 + b0 + k]],
                    buf.at[pl.ds(k * CHUNK, CHUNK)],
                    gsem,
                )

        def drain_batch(i, b0, buf, gsem):
            for k in range(8):
                pltpu.make_async_copy(
                    h_hbm.at[idx_v.at[i * 16 + b0 + k]],
                    buf.at[pl.ds(k * CHUNK, CHUNK)],
                    gsem,
                ).wait()

        def store_batch(i, b0, buf, ssem):
            base = (w * ch_per_w + i * 16 + b0) * CHUNK
            return pltpu.async_copy(buf, xj_hbm.at[pl.ds(base, 8 * CHUNK)], ssem)

        def wait_store(buf, ssem):
            pltpu.make_async_copy(buf, xj_hbm.at[pl.ds(0, 8 * CHUNK)], ssem).wait()

        def body(i, carry):
            pl.when(i > 0)(lambda: wait_store(buf_a, sa))
            gather_batch(i, 0, buf_a, ga)
            pl.when(i > 0)(lambda: wait_store(buf_b, sb))
            gather_batch(i, 8, buf_b, gb)
            drain_batch(i, 0, buf_a, ga)
            store_batch(i, 0, buf_a, sa)
            drain_batch(i, 8, buf_b, gb)
            store_batch(i, 8, buf_b, sb)
            return carry

        lax.fori_loop(0, n_it, body, 0)
        wait_store(buf_a, sa)
        wait_store(buf_b, sb)

    return gather_k


def _edge_msg_body(xj_ref, ea_ref, we_ref, be_ref, out_ref):
    t = jnp.dot(ea_ref[...], we_ref[...], preferred_element_type=jnp.float32)
    t = _leaky(t + be_ref[...])
    xj = xj_ref[...]
    c = xj.shape[1]
    acc = xj[:, 0:1] * t[:, 0:c]
    for i in range(1, c):
        acc = acc + xj[:, i : i + 1] * t[:, i * c : (i + 1) * c]
    out_ref[...] = acc


def _edge_msg(xj, ea, W_e, b_e, e_pad, block_e):
    c = xj.shape[1]
    cc = W_e.shape[1]
    grid = e_pad // block_e
    last_ea = ea.shape[0] // block_e - 1
    return pl.pallas_call(
        _edge_msg_body,
        grid=(grid,),
        in_specs=[
            pl.BlockSpec((block_e, c), lambda i: (i, 0)),
            pl.BlockSpec((block_e, W_e.shape[0]), lambda i: (jnp.minimum(i, last_ea), 0)),
            pl.BlockSpec(W_e.shape, lambda i: (0, 0)),
            pl.BlockSpec((1, cc), lambda i: (0, 0)),
        ],
        out_specs=pl.BlockSpec((block_e, c), lambda i: (i, 0)),
        out_shape=jax.ShapeDtypeStruct((e_pad, c), jnp.float32),
    )(xj, ea, W_e, b_e.reshape(1, cc))


def _make_scatter(n_pad, c, ch_per_w):
    mesh = plsc.VectorSubcoreMesh(core_axis_name="c", subcore_axis_name="s")
    rows_per_sub = n_pad // NS

    @functools.partial(
        pl.kernel,
        out_type=jax.ShapeDtypeStruct((NC, n_pad, c), jnp.float32),
        mesh=mesh,
        scratch_types=[
            pltpu.VMEM((ch_per_w, CHUNK), jnp.int32),
            pltpu.VMEM((8 * CHUNK, c), jnp.float32),
            pltpu.VMEM((8 * CHUNK, c), jnp.float32),
            pltpu.VMEM_SHARED((n_pad, c), jnp.float32),
            pltpu.SemaphoreType.DMA,
            pltpu.SemaphoreType.DMA,
        ],
        compiler_params=pltpu.CompilerParams(use_tc_tiling_on_sc=False),
    )
    def scatter_k(msg_hbm, dst_hbm, zero_hbm, agg_hbm, idx_v, buf_a, buf_b, agg_sh, la, lb):
        ci = lax.axis_index("c")
        s = lax.axis_index("s")
        w = ci * NS + s
        pltpu.sync_copy(
            zero_hbm.at[pl.ds(s * rows_per_sub, rows_per_sub)],
            agg_sh.at[pl.ds(s * rows_per_sub, rows_per_sub)],
        )
        pltpu.sync_copy(dst_hbm.at[pl.ds(w * ch_per_w, ch_per_w)], idx_v)
        plsc.subcore_barrier()
        n_it = ch_per_w // 16

        def load_batch(i, b0, buf, sem):
            base = (w * ch_per_w + i * 16 + b0) * CHUNK
            pltpu.async_copy(msg_hbm.at[pl.ds(base, 8 * CHUNK)], buf, sem)

        def wait_load(buf, sem):
            pltpu.make_async_copy(msg_hbm.at[pl.ds(0, 8 * CHUNK)], buf, sem).wait()

        def scatter_batch(i, b0, buf):
            for k in range(8):
                pltpu.sync_copy(
                    buf.at[pl.ds(k * CHUNK, CHUNK)],
                    agg_sh.at[idx_v.at[i * 16 + b0 + k]],
                    add=True,
                )

        load_batch(0, 0, buf_a, la)

        def body(i, carry):
            load_batch(i, 8, buf_b, lb)
            wait_load(buf_a, la)
            scatter_batch(i, 0, buf_a)
            nxt = lax.rem(i + 1, n_it)
            load_batch(nxt, 0, buf_a, la)
            wait_load(buf_b, lb)
            scatter_batch(i, 8, buf_b)
            return carry

        lax.fori_loop(0, n_it, body, 0)
        wait_load(buf_a, la)
        plsc.subcore_barrier()
        pltpu.sync_copy(
            agg_sh.at[pl.ds(s * rows_per_sub, rows_per_sub)],
            agg_hbm.at[ci].at[pl.ds(s * rows_per_sub, rows_per_sub)],
        )

    return scatter_k


def _head_body(a0_ref, a1_ref, h_ref, wr_ref, bc_ref, wo_ref, bo_ref, out_ref):
    z = a0_ref[...] + a1_ref[...]
    z = z + jnp.dot(h_ref[...], wr_ref[...], preferred_element_type=jnp.float32)
    z = z + bc_ref[...]
    o = jnp.dot(z, wo_ref[...], preferred_element_type=jnp.float32) + bo_ref[...]
    sp = jnp.maximum(o, 0.0) + jnp.log(1.0 + jnp.exp(-jnp.abs(o)))
    out_ref[...] = jnp.concatenate([o - sp, -sp], axis=1)


def _head(a0, a1, h, W_root, b_conv, W_out, b_out):
    n, c = h.shape
    return pl.pallas_call(
        _head_body,
        out_shape=jax.ShapeDtypeStruct((n, 2), jnp.float32),
    )(a0, a1, h, W_root, b_conv.reshape(1, c), W_out, b_out.reshape(1, 1))


def kernel(x, edge_index, edge_attr, W_in, b_in, W_e, b_e, W_root, b_conv, W_out, b_out):
    n = x.shape[0]
    e = edge_attr.shape[0]
    c = W_root.shape[0]

    n_chunks = -(-e // CHUNK)
    ch_per_w = (-(-n_chunks // NW) + 15) // 16 * 16
    e_pad = ch_per_w * NW * CHUNK
    n_pad = -(-(n + 1) // 128) * 128

    pad_src = jnp.arange(e_pad - e, dtype=jnp.int32) % n
    pad_dst = n + jnp.arange(e_pad - e, dtype=jnp.int32) % (n_pad - n)
    src = jnp.concatenate([edge_index[0], pad_src]).reshape(-1, CHUNK)
    dst = jnp.concatenate([edge_index[1], pad_dst]).reshape(-1, CHUNK)

    h = _lin_in(x, W_in, b_in)
    xj = _make_gather(n, c, ch_per_w, e_pad)(h, src)

    block_e = 2560
    assert e_pad % block_e == 0 and e % block_e == 0
    msg = _edge_msg(xj, edge_attr, W_e, b_e, e_pad, block_e)

    zero = jnp.zeros((n_pad, c), jnp.float32)
    agg = _make_scatter(n_pad, c, ch_per_w)(msg, dst, zero)

    return _head(agg[0, :n], agg[1, :n], h, W_root, b_conv, W_out, b_out)


# speedup vs baseline: 2.9245x; 2.9245x over previous
import functools

import jax
import jax.numpy as jnp
from jax import lax
from jax.experimental import pallas as pl
from jax.experimental.pallas import tpu as pltpu
from jax.experimental.pallas import tpu_sc as plsc

NC = 2
NS = 16
NW = NC * NS
CHUNK = 128


def _leaky(v):
    return jnp.where(v >= 0, v, 0.01 * v)


def _lin_in_body(x_ref, w_ref, b_ref, h_ref):
    h = jnp.dot(x_ref[...], w_ref[...], preferred_element_type=jnp.float32)
    h_ref[...] = _leaky(h + b_ref[...])


def _lin_in(x, W_in, b_in):
    n, c = x.shape[0], W_in.shape[1]
    return pl.pallas_call(
        _lin_in_body,
        out_shape=jax.ShapeDtypeStruct((n, c), jnp.float32),
    )(x, W_in, b_in.reshape(1, c))


def _make_gather(n_nodes, c, ch_per_w, e_pad):
    mesh = plsc.VectorSubcoreMesh(core_axis_name="c", subcore_axis_name="s")

    @functools.partial(
        pl.kernel,
        out_type=jax.ShapeDtypeStruct((e_pad, c), jnp.float32),
        mesh=mesh,
        scratch_types=[
            pltpu.VMEM((ch_per_w, CHUNK), jnp.int32),
            pltpu.VMEM((8 * CHUNK, c), jnp.float32),
            pltpu.VMEM((8 * CHUNK, c), jnp.float32),
            pltpu.SemaphoreType.DMA,
            pltpu.SemaphoreType.DMA,
            pltpu.SemaphoreType.DMA,
            pltpu.SemaphoreType.DMA,
        ],
        compiler_params=pltpu.CompilerParams(use_tc_tiling_on_sc=False),
    )
    def gather_k(h_hbm, src_hbm, xj_hbm, idx_v, buf_a, buf_b, ga, gb, sa, sb):
        w = lax.axis_index("c") * NS + lax.axis_index("s")
        pltpu.sync_copy(src_hbm.at[pl.ds(w * ch_per_w, ch_per_w)], idx_v)
        n_it = ch_per_w // 16

        def gather_batch(i, b0, buf, gsem):
            for k in range(8):
                pltpu.async_copy(
                    h_hbm.at[idx_v.at[i * 16 + b0 + k]],
                    buf.at[pl.ds(k * CHUNK, CHUNK)],
                    gsem,
                )

        def drain_batch(i, b0, buf, gsem):
            for k in range(8):
                pltpu.make_async_copy(
                    h_hbm.at[idx_v.at[i * 16 + b0 + k]],
                    buf.at[pl.ds(k * CHUNK, CHUNK)],
                    gsem,
                ).wait()

        def store_batch(i, b0, buf, ssem):
            base = (w * ch_per_w + i * 16 + b0) * CHUNK
            return pltpu.async_copy(buf, xj_hbm.at[pl.ds(base, 8 * CHUNK)], ssem)

        def wait_store(buf, ssem):
            pltpu.make_async_copy(buf, xj_hbm.at[pl.ds(0, 8 * CHUNK)], ssem).wait()

        def body(i, carry):
            pl.when(i > 0)(lambda: wait_store(buf_a, sa))
            gather_batch(i, 0, buf_a, ga)
            pl.when(i > 0)(lambda: wait_store(buf_b, sb))
            gather_batch(i, 8, buf_b, gb)
            drain_batch(i, 0, buf_a, ga)
            store_batch(i, 0, buf_a, sa)
            drain_batch(i, 8, buf_b, gb)
            store_batch(i, 8, buf_b, sb)
            return carry

        lax.fori_loop(0, n_it, body, 0)
        wait_store(buf_a, sa)
        wait_store(buf_b, sb)

    return gather_k


def _edge_msg_body(xj_ref, ea_ref, we_ref, be_ref, r_ref, s_ref, out_ref):
    t = jnp.dot(ea_ref[...], we_ref[...], preferred_element_type=jnp.float32)
    t = _leaky(t + be_ref[...])
    c = r_ref.shape[0]
    xj = xj_ref[...].reshape(-1, c)
    xb = jnp.dot(xj, r_ref[...], preferred_element_type=jnp.float32)
    msg = jnp.dot(xb * t, s_ref[...], preferred_element_type=jnp.float32)
    out_ref[...] = msg.reshape(out_ref.shape)


def _edge_msg(xj, ea, W_e, b_e, e_pad, block_e):
    c = xj.shape[1]
    cc = W_e.shape[1]
    grid = e_pad // block_e
    last_ea = ea.shape[0] // block_e - 1
    R = (jnp.arange(cc)[None, :] // c == jnp.arange(c)[:, None]).astype(jnp.float32)
    S = (jnp.arange(cc)[:, None] % c == jnp.arange(c)[None, :]).astype(jnp.float32)
    return pl.pallas_call(
        _edge_msg_body,
        grid=(grid,),
        in_specs=[
            pl.BlockSpec((block_e // 8, 8, c), lambda i: (i, 0, 0)),
            pl.BlockSpec((block_e, W_e.shape[0]), lambda i: (jnp.minimum(i, last_ea), 0)),
            pl.BlockSpec(W_e.shape, lambda i: (0, 0)),
            pl.BlockSpec((1, cc), lambda i: (0, 0)),
            pl.BlockSpec((c, cc), lambda i: (0, 0)),
            pl.BlockSpec((cc, c), lambda i: (0, 0)),
        ],
        out_specs=pl.BlockSpec((block_e // 8, 8, c), lambda i: (i, 0, 0)),
        out_shape=jax.ShapeDtypeStruct((e_pad // 8, 8, c), jnp.float32),
    )(xj.reshape(-1, 8, c), ea, W_e, b_e.reshape(1, cc), R, S)


def _make_scatter(n_pad, c, ch_per_w):
    mesh = plsc.VectorSubcoreMesh(core_axis_name="c", subcore_axis_name="s")
    rows_per_sub = n_pad // NS

    @functools.partial(
        pl.kernel,
        out_type=jax.ShapeDtypeStruct((NC, n_pad, c), jnp.float32),
        mesh=mesh,
        scratch_types=[
            pltpu.VMEM((ch_per_w, CHUNK), jnp.int32),
            pltpu.VMEM((8 * CHUNK, c), jnp.float32),
            pltpu.VMEM((8 * CHUNK, c), jnp.float32),
            pltpu.VMEM_SHARED((n_pad, c), jnp.float32),
            pltpu.SemaphoreType.DMA,
            pltpu.SemaphoreType.DMA,
        ],
        compiler_params=pltpu.CompilerParams(use_tc_tiling_on_sc=False),
    )
    def scatter_k(msg_hbm, dst_hbm, zero_hbm, agg_hbm, idx_v, buf_a, buf_b, agg_sh, la, lb):
        ci = lax.axis_index("c")
        s = lax.axis_index("s")
        w = ci * NS + s
        pltpu.sync_copy(
            zero_hbm.at[pl.ds(s * rows_per_sub, rows_per_sub)],
            agg_sh.at[pl.ds(s * rows_per_sub, rows_per_sub)],
        )
        pltpu.sync_copy(dst_hbm.at[pl.ds(w * ch_per_w, ch_per_w)], idx_v)
        plsc.subcore_barrier()
        n_it = ch_per_w // 16

        def load_batch(i, b0, buf, sem):
            base = (w * ch_per_w + i * 16 + b0) * CHUNK
            pltpu.async_copy(msg_hbm.at[pl.ds(base, 8 * CHUNK)], buf, sem)

        def wait_load(buf, sem):
            pltpu.make_async_copy(msg_hbm.at[pl.ds(0, 8 * CHUNK)], buf, sem).wait()

        def scatter_batch(i, b0, buf):
            for k in range(8):
                pltpu.sync_copy(
                    buf.at[pl.ds(k * CHUNK, CHUNK)],
                    agg_sh.at[idx_v.at[i * 16 + b0 + k]],
                    add=True,
                )

        load_batch(0, 0, buf_a, la)

        def body(i, carry):
            load_batch(i, 8, buf_b, lb)
            wait_load(buf_a, la)
            scatter_batch(i, 0, buf_a)
            nxt = lax.rem(i + 1, n_it)
            load_batch(nxt, 0, buf_a, la)
            wait_load(buf_b, lb)
            scatter_batch(i, 8, buf_b)
            return carry

        lax.fori_loop(0, n_it, body, 0)
        wait_load(buf_a, la)
        plsc.subcore_barrier()
        pltpu.sync_copy(
            agg_sh.at[pl.ds(s * rows_per_sub, rows_per_sub)],
            agg_hbm.at[ci].at[pl.ds(s * rows_per_sub, rows_per_sub)],
        )

    return scatter_k


def _head_body(a0_ref, a1_ref, h_ref, wr_ref, bc_ref, wo_ref, bo_ref, out_ref):
    z = a0_ref[...] + a1_ref[...]
    z = z + jnp.dot(h_ref[...], wr_ref[...], preferred_element_type=jnp.float32)
    z = z + bc_ref[...]
    o = jnp.dot(z, wo_ref[...], preferred_element_type=jnp.float32) + bo_ref[...]
    sp = jnp.maximum(o, 0.0) + jnp.log(1.0 + jnp.exp(-jnp.abs(o)))
    out_ref[...] = jnp.concatenate([o - sp, -sp], axis=1)


def _head(a0, a1, h, W_root, b_conv, W_out, b_out):
    n, c = h.shape
    return pl.pallas_call(
        _head_body,
        out_shape=jax.ShapeDtypeStruct((n, 2), jnp.float32),
    )(a0, a1, h, W_root, b_conv.reshape(1, c), W_out, b_out.reshape(1, 1))


def kernel(x, edge_index, edge_attr, W_in, b_in, W_e, b_e, W_root, b_conv, W_out, b_out):
    n = x.shape[0]
    e = edge_attr.shape[0]
    c = W_root.shape[0]

    n_chunks = -(-e // CHUNK)
    ch_per_w = (-(-n_chunks // NW) + 15) // 16 * 16
    e_pad = ch_per_w * NW * CHUNK
    n_pad = -(-(n + 1) // 128) * 128

    pad_src = jnp.arange(e_pad - e, dtype=jnp.int32) % n
    pad_dst = n + jnp.arange(e_pad - e, dtype=jnp.int32) % (n_pad - n)
    src = jnp.concatenate([edge_index[0], pad_src]).reshape(-1, CHUNK)
    dst = jnp.concatenate([edge_index[1], pad_dst]).reshape(-1, CHUNK)

    h = _lin_in(x, W_in, b_in)
    xj = _make_gather(n, c, ch_per_w, e_pad)(h, src)

    block_e = 2560
    assert e_pad % block_e == 0 and e % block_e == 0
    msg = _edge_msg(xj, edge_attr, W_e, b_e, e_pad, block_e).reshape(e_pad, c)

    zero = jnp.zeros((n_pad, c), jnp.float32)
    agg = _make_scatter(n_pad, c, ch_per_w)(msg, dst, zero)

    return _head(agg[0, :n], agg[1, :n], h, W_root, b_conv, W_out, b_out)


# speedup vs baseline: 4.3397x; 1.4839x over previous
import functools

import jax
import jax.numpy as jnp
from jax import lax
from jax.experimental import pallas as pl
from jax.experimental.pallas import tpu as pltpu
from jax.experimental.pallas import tpu_sc as plsc

NC = 2
NS = 16
NW = NC * NS
CHUNK = 128


def _leaky(v):
    return jnp.where(v >= 0, v, 0.01 * v)


def _lin_in_body(x_ref, w_ref, b_ref, h_ref):
    h = jnp.dot(x_ref[...], w_ref[...], preferred_element_type=jnp.float32)
    h_ref[...] = _leaky(h + b_ref[...])


def _lin_in(x, W_in, b_in):
    n, c = x.shape[0], W_in.shape[1]
    return pl.pallas_call(
        _lin_in_body,
        out_shape=jax.ShapeDtypeStruct((n, c), jnp.float32),
    )(x, W_in, b_in.reshape(1, c))


def _make_gather(n_nodes, c, ch_per_w, e_pad):
    mesh = plsc.VectorSubcoreMesh(core_axis_name="c", subcore_axis_name="s")

    @functools.partial(
        pl.kernel,
        out_type=jax.ShapeDtypeStruct((e_pad, c), jnp.float32),
        mesh=mesh,
        scratch_types=[
            pltpu.VMEM((ch_per_w, CHUNK), jnp.int32),
            pltpu.VMEM((8 * CHUNK, c), jnp.float32),
            pltpu.VMEM((8 * CHUNK, c), jnp.float32),
            pltpu.SemaphoreType.DMA,
            pltpu.SemaphoreType.DMA,
            pltpu.SemaphoreType.DMA,
            pltpu.SemaphoreType.DMA,
        ],
        compiler_params=pltpu.CompilerParams(use_tc_tiling_on_sc=False),
    )
    def gather_k(h_hbm, src_hbm, xj_hbm, idx_v, buf_a, buf_b, ga, gb, sa, sb):
        w = lax.axis_index("c") * NS + lax.axis_index("s")
        pltpu.sync_copy(src_hbm.at[pl.ds(w * ch_per_w, ch_per_w)], idx_v)
        n_it = ch_per_w // 16

        def gather_batch(i, b0, buf, gsem):
            for k in range(8):
                pltpu.async_copy(
                    h_hbm.at[idx_v.at[i * 16 + b0 + k]],
                    buf.at[pl.ds(k * CHUNK, CHUNK)],
                    gsem,
                )

        def drain_batch(i, b0, buf, gsem):
            for k in range(8):
                pltpu.make_async_copy(
                    h_hbm.at[idx_v.at[i * 16 + b0 + k]],
                    buf.at[pl.ds(k * CHUNK, CHUNK)],
                    gsem,
                ).wait()

        def store_batch(i, b0, buf, ssem):
            base = (w * ch_per_w + i * 16 + b0) * CHUNK
            return pltpu.async_copy(buf, xj_hbm.at[pl.ds(base, 8 * CHUNK)], ssem)

        def wait_store(buf, ssem):
            pltpu.make_async_copy(buf, xj_hbm.at[pl.ds(0, 8 * CHUNK)], ssem).wait()

        def body(i, carry):
            pl.when(i > 0)(lambda: wait_store(buf_a, sa))
            gather_batch(i, 0, buf_a, ga)
            pl.when(i > 0)(lambda: wait_store(buf_b, sb))
            gather_batch(i, 8, buf_b, gb)
            drain_batch(i, 0, buf_a, ga)
            store_batch(i, 0, buf_a, sa)
            drain_batch(i, 8, buf_b, gb)
            store_batch(i, 8, buf_b, sb)
            return carry

        lax.fori_loop(0, n_it, body, 0)
        wait_store(buf_a, sa)
        wait_store(buf_b, sb)

    return gather_k


def _edge_msg_body(xj_ref, ea_ref, we_ref, be_ref, r_ref, s_ref, out_ref):
    t = jnp.dot(ea_ref[...], we_ref[...], preferred_element_type=jnp.float32)
    t = _leaky(t + be_ref[...])
    xb = jnp.dot(xj_ref[...], r_ref[...], preferred_element_type=jnp.float32)
    out_ref[...] = jnp.dot(xb * t, s_ref[...], preferred_element_type=jnp.float32)


def _edge_msg(xj, ea, W_e, b_e, e_pad, block_e):
    c = xj.shape[1]
    cc = W_e.shape[1]
    d = W_e.shape[0]
    grid = e_pad // block_e
    rows = block_e // 8
    last_ea = ea.shape[0] // block_e - 1
    R = (jnp.arange(cc)[None, :] // c == jnp.arange(c)[:, None]).astype(jnp.float32)
    S = (jnp.arange(cc)[:, None] % c == jnp.arange(c)[None, :]).astype(jnp.float32)
    eye8 = jnp.eye(8, dtype=jnp.float32)
    Wb = jnp.kron(eye8, W_e)
    Rb = jnp.kron(eye8, R)
    Sb = jnp.kron(eye8, S)
    bb = jnp.tile(b_e, 8).reshape(1, 8 * cc)
    return pl.pallas_call(
        _edge_msg_body,
        grid=(grid,),
        in_specs=[
            pl.BlockSpec((rows, 8 * c), lambda i: (i, 0)),
            pl.BlockSpec((rows, 8 * d), lambda i: (jnp.minimum(i, last_ea), 0)),
            pl.BlockSpec((8 * d, 8 * cc), lambda i: (0, 0)),
            pl.BlockSpec((1, 8 * cc), lambda i: (0, 0)),
            pl.BlockSpec((8 * c, 8 * cc), lambda i: (0, 0)),
            pl.BlockSpec((8 * cc, 8 * c), lambda i: (0, 0)),
        ],
        out_specs=pl.BlockSpec((rows, 8 * c), lambda i: (i, 0)),
        out_shape=jax.ShapeDtypeStruct((e_pad // 8, 8 * c), jnp.float32),
    )(xj.reshape(-1, 8 * c), ea.reshape(-1, 8 * d), Wb, bb, Rb, Sb)


def _make_scatter(n_pad, c, ch_per_w):
    mesh = plsc.VectorSubcoreMesh(core_axis_name="c", subcore_axis_name="s")
    rows_per_sub = n_pad // NS

    @functools.partial(
        pl.kernel,
        out_type=jax.ShapeDtypeStruct((NC, n_pad, c), jnp.float32),
        mesh=mesh,
        scratch_types=[
            pltpu.VMEM((ch_per_w, CHUNK), jnp.int32),
            pltpu.VMEM((8 * CHUNK, c), jnp.float32),
            pltpu.VMEM((8 * CHUNK, c), jnp.float32),
            pltpu.VMEM_SHARED((n_pad, c), jnp.float32),
            pltpu.SemaphoreType.DMA,
            pltpu.SemaphoreType.DMA,
        ],
        compiler_params=pltpu.CompilerParams(use_tc_tiling_on_sc=False),
    )
    def scatter_k(msg_hbm, dst_hbm, zero_hbm, agg_hbm, idx_v, buf_a, buf_b, agg_sh, la, lb):
        ci = lax.axis_index("c")
        s = lax.axis_index("s")
        w = ci * NS + s
        pltpu.sync_copy(
            zero_hbm.at[pl.ds(s * rows_per_sub, rows_per_sub)],
            agg_sh.at[pl.ds(s * rows_per_sub, rows_per_sub)],
        )
        pltpu.sync_copy(dst_hbm.at[pl.ds(w * ch_per_w, ch_per_w)], idx_v)
        plsc.subcore_barrier()
        n_it = ch_per_w // 16

        def load_batch(i, b0, buf, sem):
            base = (w * ch_per_w + i * 16 + b0) * CHUNK
            pltpu.async_copy(msg_hbm.at[pl.ds(base, 8 * CHUNK)], buf, sem)

        def wait_load(buf, sem):
            pltpu.make_async_copy(msg_hbm.at[pl.ds(0, 8 * CHUNK)], buf, sem).wait()

        def scatter_batch(i, b0, buf):
            for k in range(8):
                pltpu.sync_copy(
                    buf.at[pl.ds(k * CHUNK, CHUNK)],
                    agg_sh.at[idx_v.at[i * 16 + b0 + k]],
                    add=True,
                )

        load_batch(0, 0, buf_a, la)

        def body(i, carry):
            load_batch(i, 8, buf_b, lb)
            wait_load(buf_a, la)
            scatter_batch(i, 0, buf_a)
            nxt = lax.rem(i + 1, n_it)
            load_batch(nxt, 0, buf_a, la)
            wait_load(buf_b, lb)
            scatter_batch(i, 8, buf_b)
            return carry

        lax.fori_loop(0, n_it, body, 0)
        wait_load(buf_a, la)
        plsc.subcore_barrier()
        pltpu.sync_copy(
            agg_sh.at[pl.ds(s * rows_per_sub, rows_per_sub)],
            agg_hbm.at[ci].at[pl.ds(s * rows_per_sub, rows_per_sub)],
        )

    return scatter_k


def _head_body(a0_ref, a1_ref, h_ref, wr_ref, bc_ref, wo_ref, bo_ref, out_ref):
    z = a0_ref[...] + a1_ref[...]
    z = z + jnp.dot(h_ref[...], wr_ref[...], preferred_element_type=jnp.float32)
    z = z + bc_ref[...]
    o = jnp.dot(z, wo_ref[...], preferred_element_type=jnp.float32) + bo_ref[...]
    sp = jnp.maximum(o, 0.0) + jnp.log(1.0 + jnp.exp(-jnp.abs(o)))
    out_ref[...] = jnp.concatenate([o - sp, -sp], axis=1)


def _head(a0, a1, h, W_root, b_conv, W_out, b_out):
    n, c = h.shape
    return pl.pallas_call(
        _head_body,
        out_shape=jax.ShapeDtypeStruct((n, 2), jnp.float32),
    )(a0, a1, h, W_root, b_conv.reshape(1, c), W_out, b_out.reshape(1, 1))


def kernel(x, edge_index, edge_attr, W_in, b_in, W_e, b_e, W_root, b_conv, W_out, b_out):
    n = x.shape[0]
    e = edge_attr.shape[0]
    c = W_root.shape[0]

    n_chunks = -(-e // CHUNK)
    ch_per_w = (-(-n_chunks // NW) + 15) // 16 * 16
    e_pad = ch_per_w * NW * CHUNK
    n_pad = -(-(n + 1) // 128) * 128

    pad_src = jnp.arange(e_pad - e, dtype=jnp.int32) % n
    pad_dst = n + jnp.arange(e_pad - e, dtype=jnp.int32) % (n_pad - n)
    src = jnp.concatenate([edge_index[0], pad_src]).reshape(-1, CHUNK)
    dst = jnp.concatenate([edge_index[1], pad_dst]).reshape(-1, CHUNK)

    h = _lin_in(x, W_in, b_in)
    xj = _make_gather(n, c, ch_per_w, e_pad)(h, src)

    block_e = 2560
    assert e_pad % block_e == 0 and e % block_e == 0
    msg = _edge_msg(xj, edge_attr, W_e, b_e, e_pad, block_e).reshape(e_pad, c)

    zero = jnp.zeros((n_pad, c), jnp.float32)
    agg = _make_scatter(n_pad, c, ch_per_w)(msg, dst, zero)

    return _head(agg[0, :n], agg[1, :n], h, W_root, b_conv, W_out, b_out)
